# Initial kernel scaffold; baseline (speedup 1.0000x reference)
import functools

import jax
import jax.numpy as jnp
from jax import lax
from jax.experimental import pallas as pl
from jax.experimental.pallas import tpu as pltpu
from jax.experimental.pallas import tpu_sc as plsc

N = 10000
NPAD = 10240
E = 320000
NG = 64

_NC = 2
_NS = 16
_NW = _NC * _NS
_EPW = E // _NW
_EC = 80
_NCHUNK = _EPW // _EC
_RPS = NPAD // _NS

_mesh = plsc.VectorSubcoreMesh(core_axis_name="c", subcore_axis_name="s")


@functools.partial(
    pl.kernel,
    out_type=jax.ShapeDtypeStruct((_NC, NPAD), jnp.float32),
    mesh=_mesh,
    scratch_types=[
        pltpu.VMEM((_EC,), jnp.int32),
        pltpu.VMEM((_EC,), jnp.float32),
        pltpu.VMEM((_RPS,), jnp.float32),
        pltpu.VMEM_SHARED((NPAD,), jnp.float32),
    ],
)
def _sc_degree(dst_hbm, out_hbm, idx_v, ones_v, zrow_v, acc_sh):
    c = lax.axis_index("c")
    s = lax.axis_index("s")
    w = s * _NC + c
    one16 = jnp.ones((16,), jnp.float32)
    zero16 = jnp.zeros((16,), jnp.float32)
    for i in range(_EC // 16):
        ones_v[pl.ds(i * 16, 16)] = one16

    def _z(i, _):
        zrow_v[pl.ds(i * 16, 16)] = zero16
        return 0

    lax.fori_loop(0, _RPS // 16, _z, 0)
    pltpu.sync_copy(zrow_v, acc_sh.at[pl.ds(s * _RPS, _RPS)])
    plsc.subcore_barrier()

    def _body(k, _):
        base = w * _EPW + k * _EC
        pltpu.sync_copy(dst_hbm.at[pl.ds(base, _EC)], idx_v)
        pltpu.sync_copy(ones_v, acc_sh.at[idx_v], add=True)
        return 0

    lax.fori_loop(0, _NCHUNK, _body, 0)
    plsc.subcore_barrier()
    pltpu.sync_copy(acc_sh.at[pl.ds(s * _RPS, _RPS)],
                    out_hbm.at[c, pl.ds(s * _RPS, _RPS)])


def _make_spmm(F):
    @functools.partial(
        pl.kernel,
        out_type=jax.ShapeDtypeStruct((_NC, NPAD, F), jnp.float32),
        mesh=_mesh,
        scratch_types=[
            pltpu.VMEM((_EC,), jnp.int32),
            pltpu.VMEM((_EC,), jnp.int32),
            pltpu.VMEM((_EC, F), jnp.float32),
            pltpu.VMEM_SHARED((NPAD, F), jnp.float32),
            pltpu.SemaphoreType.DMA,
        ],
    )
    def _spmm(src_hbm, dst_hbm, hs_hbm, out_hbm, srcv, dstv, rows_v, acc_sh, sem):
        c = lax.axis_index("c")
        s = lax.axis_index("s")
        w = s * _NC + c
        zero16 = jnp.zeros((16,), jnp.float32)

        def _z(r, _):
            for j in range(F // 16):
                rows_v[r, pl.ds(j * 16, 16)] = zero16
            return 0

        lax.fori_loop(0, _EC, _z, 0)
        for t in range(_RPS // _EC):
            pltpu.sync_copy(rows_v, acc_sh.at[pl.ds(s * _RPS + t * _EC, _EC)])
        plsc.subcore_barrier()

        def _body(k, _):
            base = w * _EPW + k * _EC
            pltpu.sync_copy(src_hbm.at[pl.ds(base, _EC)], srcv)
            pltpu.sync_copy(dst_hbm.at[pl.ds(base, _EC)], dstv)
            pltpu.async_copy(hs_hbm.at[srcv], rows_v, sem).wait()
            pltpu.sync_copy(rows_v, acc_sh.at[dstv], add=True)
            return 0

        lax.fori_loop(0, _NCHUNK, _body, 0)
        plsc.subcore_barrier()
        pltpu.sync_copy(acc_sh.at[pl.ds(s * _RPS, _RPS)],
                        out_hbm.at[c, pl.ds(s * _RPS, _RPS)])

    return _spmm


_spmm32 = _make_spmm(32)
_spmm64 = _make_spmm(64)
_spmm128 = _make_spmm(128)


def _tc_first(xp, w1, degp):
    def body(x_ref, w1_ref, degp_ref, dinv_ref, hs_ref):
        deg = degp_ref[0] + degp_ref[1] + 1.0
        dinv = lax.rsqrt(deg)
        dinv_ref[...] = dinv
        hs_ref[...] = dinv * jnp.dot(x_ref[...], w1_ref[...],
                                     preferred_element_type=jnp.float32)

    return pl.pallas_call(
        body,
        out_shape=(
            jax.ShapeDtypeStruct((NPAD, 1), jnp.float32),
            jax.ShapeDtypeStruct((NPAD, w1.shape[1]), jnp.float32),
        ),
    )(xp, w1, degp)


def _tc_mid(accp, hs, dinv, b2d, w):
    def body(accp_ref, hs_ref, dinv_ref, b_ref, w_ref, out_ref):
        a = accp_ref[0] + accp_ref[1] + hs_ref[...]
        h = dinv_ref[...] * a + b_ref[...]
        h = jnp.maximum(h, 0.0)
        out_ref[...] = dinv_ref[...] * jnp.dot(h, w_ref[...],
                                               preferred_element_type=jnp.float32)

    return pl.pallas_call(
        body,
        out_shape=jax.ShapeDtypeStruct((NPAD, w.shape[1]), jnp.float32),
    )(accp, hs, dinv, b2d, w)


_RB = 1024
_GRID = NPAD // _RB


def _tc_head(accp, hs3, dinv, b3_2d, batch2d, wl1, bl1, wl2, bl2, wl3, bl3):
    def body(accp_ref, hs_ref, dinv_ref, b3_ref, batch_ref, wl1_ref, bl1_ref,
             wl2_ref, bl2_ref, wl3_ref, bl3_ref, out_ref, sums, cnts):
        i = pl.program_id(0)
        h3 = dinv_ref[...] * (accp_ref[0] + accp_ref[1] + hs_ref[...]) + b3_ref[...]
        z = jnp.maximum(jnp.dot(h3, wl1_ref[...],
                                preferred_element_type=jnp.float32) + bl1_ref[...], 0.0)
        z = jnp.maximum(jnp.dot(z, wl2_ref[...],
                                preferred_element_type=jnp.float32) + bl2_ref[...], 0.0)
        gids = lax.broadcasted_iota(jnp.int32, (NG, _RB), 0)
        onehot = (gids == jnp.broadcast_to(batch_ref[...], (NG, _RB))).astype(jnp.float32)
        psum = jnp.dot(onehot, z, preferred_element_type=jnp.float32)
        pcnt = jnp.sum(onehot, axis=1, keepdims=True)

        @pl.when(i == 0)
        def _():
            sums[...] = jnp.zeros_like(sums)
            cnts[...] = jnp.zeros_like(cnts)

        sums[...] += psum
        cnts[...] += pcnt

        @pl.when(i == _GRID - 1)
        def _():
            mean = sums[...] / jnp.maximum(cnts[...], 1.0)
            out_ref[...] = jnp.dot(mean, wl3_ref[...],
                                   preferred_element_type=jnp.float32) + bl3_ref[...]

    h2 = wl2.shape[0]
    h3w = wl2.shape[1]
    return pl.pallas_call(
        body,
        grid=(_GRID,),
        in_specs=[
            pl.BlockSpec((_NC, _RB, 128), lambda i: (0, i, 0)),
            pl.BlockSpec((_RB, 128), lambda i: (i, 0)),
            pl.BlockSpec((_RB, 1), lambda i: (i, 0)),
            pl.BlockSpec((1, 128), lambda i: (0, 0)),
            pl.BlockSpec((1, _RB), lambda i: (0, i)),
            pl.BlockSpec((128, h2), lambda i: (0, 0)),
            pl.BlockSpec((1, h2), lambda i: (0, 0)),
            pl.BlockSpec((h2, h3w), lambda i: (0, 0)),
            pl.BlockSpec((1, h3w), lambda i: (0, 0)),
            pl.BlockSpec((h3w, 4), lambda i: (0, 0)),
            pl.BlockSpec((1, 4), lambda i: (0, 0)),
        ],
        out_specs=pl.BlockSpec((NG, 4), lambda i: (0, 0)),
        out_shape=jax.ShapeDtypeStruct((NG, 4), jnp.float32),
        scratch_shapes=[
            pltpu.VMEM((NG, h3w), jnp.float32),
            pltpu.VMEM((NG, 1), jnp.float32),
        ],
    )(accp, hs3, dinv, b3_2d, batch2d, wl1, bl1, wl2, bl2, wl3, bl3)


def kernel(x, edge_index, batch, W1, b1, W2, b2, W3, b3,
           Wl1, bl1, Wl2, bl2, Wl3, bl3):
    src = edge_index[0].astype(jnp.int32)
    dst = edge_index[1].astype(jnp.int32)
    xp = jnp.pad(x, ((0, NPAD - N), (0, 0)))
    batch2d = jnp.pad(batch.astype(jnp.int32), (0, NPAD - N),
                      constant_values=NG).reshape(1, NPAD)

    degp = _sc_degree(dst).reshape(_NC, NPAD, 1)
    dinv, hs1 = _tc_first(xp, W1, degp)
    acc1 = _spmm32(src, dst, hs1)
    hs2 = _tc_mid(acc1, hs1, dinv, b1.reshape(1, -1), W2)
    acc2 = _spmm64(src, dst, hs2)
    hs3 = _tc_mid(acc2, hs2, dinv, b2.reshape(1, -1), W3)
    acc3 = _spmm128(src, dst, hs3)
    return _tc_head(acc3, hs3, dinv, b3.reshape(1, -1), batch2d,
                    Wl1, bl1.reshape(1, -1), Wl2, bl2.reshape(1, -1),
                    Wl3, bl3.reshape(1, -1))


# speedup vs baseline: 12.4821x; 12.4821x over previous
import functools

import jax
import jax.numpy as jnp
from jax import lax
from jax.experimental import pallas as pl
from jax.experimental.pallas import tpu as pltpu
from jax.experimental.pallas import tpu_sc as plsc

N = 10000
NPAD = 10240
E = 320000
NG = 64

_NC = 2
_NS = 16
_NW = _NC * _NS
_EPW = E // _NW
_EC = 80
_NCHUNK = _EPW // _EC
_RPS = NPAD // _NS

_mesh = plsc.VectorSubcoreMesh(core_axis_name="c", subcore_axis_name="s")
_sc_params = pltpu.CompilerParams(use_tc_tiling_on_sc=False)


@functools.partial(
    pl.kernel,
    out_type=jax.ShapeDtypeStruct((_NC, NPAD), jnp.float32),
    mesh=_mesh,
    scratch_types=[
        pltpu.VMEM((_EC,), jnp.int32),
        pltpu.VMEM((_EC,), jnp.float32),
        pltpu.VMEM((_RPS,), jnp.float32),
        pltpu.VMEM_SHARED((NPAD,), jnp.float32),
    ],
    compiler_params=_sc_params,
)
def _sc_degree(dst_hbm, out_hbm, idx_v, ones_v, zrow_v, acc_sh):
    c = lax.axis_index("c")
    s = lax.axis_index("s")
    w = s * _NC + c
    one16 = jnp.ones((16,), jnp.float32)
    zero16 = jnp.zeros((16,), jnp.float32)
    for i in range(_EC // 16):
        ones_v[pl.ds(i * 16, 16)] = one16

    def _z(i, _):
        zrow_v[pl.ds(i * 16, 16)] = zero16
        return 0

    lax.fori_loop(0, _RPS // 16, _z, 0)
    pltpu.sync_copy(zrow_v, acc_sh.at[pl.ds(s * _RPS, _RPS)])
    plsc.subcore_barrier()

    def _body(k, _):
        base = w * _EPW + k * _EC
        pltpu.sync_copy(dst_hbm.at[pl.ds(base, _EC)], idx_v)
        pltpu.sync_copy(ones_v, acc_sh.at[idx_v], add=True)
        return 0

    lax.fori_loop(0, _NCHUNK, _body, 0)
    plsc.subcore_barrier()
    pltpu.sync_copy(acc_sh.at[pl.ds(s * _RPS, _RPS)],
                    out_hbm.at[c, pl.ds(s * _RPS, _RPS)])


def _make_spmm(F):
    @functools.partial(
        pl.kernel,
        out_type=jax.ShapeDtypeStruct((_NC, NPAD, F), jnp.float32),
        mesh=_mesh,
        scratch_types=[
            pltpu.VMEM((_EC,), jnp.int32),
            pltpu.VMEM((_EC,), jnp.int32),
            pltpu.VMEM((_EC, F), jnp.float32),
            pltpu.VMEM_SHARED((NPAD, F), jnp.float32),
            pltpu.SemaphoreType.DMA,
        ],
        compiler_params=_sc_params,
    )
    def _spmm(src_hbm, dst_hbm, hs_hbm, out_hbm, srcv, dstv, rows_v, acc_sh, sem):
        c = lax.axis_index("c")
        s = lax.axis_index("s")
        w = s * _NC + c
        zero16 = jnp.zeros((16,), jnp.float32)

        def _z(r, _):
            for j in range(F // 16):
                rows_v[r, pl.ds(j * 16, 16)] = zero16
            return 0

        lax.fori_loop(0, _EC, _z, 0)
        for t in range(_RPS // _EC):
            pltpu.sync_copy(rows_v, acc_sh.at[pl.ds(s * _RPS + t * _EC, _EC)])
        plsc.subcore_barrier()

        def _body(k, _):
            base = w * _EPW + k * _EC
            pltpu.sync_copy(src_hbm.at[pl.ds(base, _EC)], srcv)
            pltpu.sync_copy(dst_hbm.at[pl.ds(base, _EC)], dstv)
            pltpu.async_copy(hs_hbm.at[srcv], rows_v, sem).wait()
            pltpu.sync_copy(rows_v, acc_sh.at[dstv], add=True)
            return 0

        lax.fori_loop(0, _NCHUNK, _body, 0)
        plsc.subcore_barrier()
        pltpu.sync_copy(acc_sh.at[pl.ds(s * _RPS, _RPS)],
                        out_hbm.at[c, pl.ds(s * _RPS, _RPS)])

    return _spmm


_spmm32 = _make_spmm(32)
_spmm64 = _make_spmm(64)
_spmm128 = _make_spmm(128)


def _tc_first(xp, w1, degp):
    def body(x_ref, w1_ref, degp_ref, dinv_ref, hs_ref):
        deg = degp_ref[0] + degp_ref[1] + 1.0
        dinv = lax.rsqrt(deg)
        dinv_ref[...] = dinv
        hs_ref[...] = dinv * jnp.dot(x_ref[...], w1_ref[...],
                                     preferred_element_type=jnp.float32)

    return pl.pallas_call(
        body,
        out_shape=(
            jax.ShapeDtypeStruct((NPAD, 1), jnp.float32),
            jax.ShapeDtypeStruct((NPAD, w1.shape[1]), jnp.float32),
        ),
    )(xp, w1, degp)


def _tc_mid(accp, hs, dinv, b2d, w):
    def body(accp_ref, hs_ref, dinv_ref, b_ref, w_ref, out_ref):
        a = accp_ref[0] + accp_ref[1] + hs_ref[...]
        h = dinv_ref[...] * a + b_ref[...]
        h = jnp.maximum(h, 0.0)
        out_ref[...] = dinv_ref[...] * jnp.dot(h, w_ref[...],
                                               preferred_element_type=jnp.float32)

    return pl.pallas_call(
        body,
        out_shape=jax.ShapeDtypeStruct((NPAD, w.shape[1]), jnp.float32),
    )(accp, hs, dinv, b2d, w)


_RB = 1024
_GRID = NPAD // _RB


def _tc_head(accp, hs3, dinv, b3_2d, batch2d, wl1, bl1, wl2, bl2, wl3, bl3):
    def body(accp_ref, hs_ref, dinv_ref, b3_ref, batch_ref, wl1_ref, bl1_ref,
             wl2_ref, bl2_ref, wl3_ref, bl3_ref, out_ref, sums, cnts):
        i = pl.program_id(0)
        h3 = dinv_ref[...] * (accp_ref[0] + accp_ref[1] + hs_ref[...]) + b3_ref[...]
        z = jnp.maximum(jnp.dot(h3, wl1_ref[...],
                                preferred_element_type=jnp.float32) + bl1_ref[...], 0.0)
        z = jnp.maximum(jnp.dot(z, wl2_ref[...],
                                preferred_element_type=jnp.float32) + bl2_ref[...], 0.0)
        gids = lax.broadcasted_iota(jnp.int32, (NG, _RB), 0)
        onehot = (gids == jnp.broadcast_to(batch_ref[...], (NG, _RB))).astype(jnp.float32)
        psum = jnp.dot(onehot, z, preferred_element_type=jnp.float32)
        pcnt = jnp.sum(onehot, axis=1, keepdims=True)

        @pl.when(i == 0)
        def _():
            sums[...] = jnp.zeros_like(sums)
            cnts[...] = jnp.zeros_like(cnts)

        sums[...] += psum
        cnts[...] += pcnt

        @pl.when(i == _GRID - 1)
        def _():
            mean = sums[...] / jnp.maximum(cnts[...], 1.0)
            out_ref[...] = jnp.dot(mean, wl3_ref[...],
                                   preferred_element_type=jnp.float32) + bl3_ref[...]

    h2 = wl2.shape[0]
    h3w = wl2.shape[1]
    return pl.pallas_call(
        body,
        grid=(_GRID,),
        in_specs=[
            pl.BlockSpec((_NC, _RB, 128), lambda i: (0, i, 0)),
            pl.BlockSpec((_RB, 128), lambda i: (i, 0)),
            pl.BlockSpec((_RB, 1), lambda i: (i, 0)),
            pl.BlockSpec((1, 128), lambda i: (0, 0)),
            pl.BlockSpec((1, _RB), lambda i: (0, i)),
            pl.BlockSpec((128, h2), lambda i: (0, 0)),
            pl.BlockSpec((1, h2), lambda i: (0, 0)),
            pl.BlockSpec((h2, h3w), lambda i: (0, 0)),
            pl.BlockSpec((1, h3w), lambda i: (0, 0)),
            pl.BlockSpec((h3w, 4), lambda i: (0, 0)),
            pl.BlockSpec((1, 4), lambda i: (0, 0)),
        ],
        out_specs=pl.BlockSpec((NG, 4), lambda i: (0, 0)),
        out_shape=jax.ShapeDtypeStruct((NG, 4), jnp.float32),
        scratch_shapes=[
            pltpu.VMEM((NG, h3w), jnp.float32),
            pltpu.VMEM((NG, 1), jnp.float32),
        ],
    )(accp, hs3, dinv, b3_2d, batch2d, wl1, bl1, wl2, bl2, wl3, bl3)


def kernel(x, edge_index, batch, W1, b1, W2, b2, W3, b3,
           Wl1, bl1, Wl2, bl2, Wl3, bl3):
    src = edge_index[0].astype(jnp.int32)
    dst = edge_index[1].astype(jnp.int32)
    xp = jnp.pad(x, ((0, NPAD - N), (0, 0)))
    batch2d = jnp.pad(batch.astype(jnp.int32), (0, NPAD - N),
                      constant_values=NG).reshape(1, NPAD)

    degp = _sc_degree(dst).reshape(_NC, NPAD, 1)
    dinv, hs1 = _tc_first(xp, W1, degp)
    acc1 = _spmm32(src, dst, hs1)
    hs2 = _tc_mid(acc1, hs1, dinv, b1.reshape(1, -1), W2)
    acc2 = _spmm64(src, dst, hs2)
    hs3 = _tc_mid(acc2, hs2, dinv, b2.reshape(1, -1), W3)
    acc3 = _spmm128(src, dst, hs3)
    return _tc_head(acc3, hs3, dinv, b3.reshape(1, -1), batch2d,
                    Wl1, bl1.reshape(1, -1), Wl2, bl2.reshape(1, -1),
                    Wl3, bl3.reshape(1, -1))


# speedup vs baseline: 14.5762x; 1.1678x over previous
import functools

import jax
import jax.numpy as jnp
from jax import lax
from jax.experimental import pallas as pl
from jax.experimental.pallas import tpu as pltpu
from jax.experimental.pallas import tpu_sc as plsc

N = 10000
NPAD = 10240
E = 320000
NG = 64

_NC = 2
_NS = 16
_NW = _NC * _NS
_ECH = 128
_EROWS = 2560
_EPAD = _ECH * _EROWS
_RPW = _EROWS // _NW
_RPS = NPAD // _NS

_mesh = plsc.VectorSubcoreMesh(core_axis_name="c", subcore_axis_name="s")
_sc_params = pltpu.CompilerParams(use_tc_tiling_on_sc=False)


@functools.partial(
    pl.kernel,
    out_type=jax.ShapeDtypeStruct((_NC, NPAD), jnp.float32),
    mesh=_mesh,
    scratch_types=[
        pltpu.VMEM((_RPW, _ECH), jnp.int32),
        pltpu.VMEM((_ECH,), jnp.float32),
        pltpu.VMEM((_RPS,), jnp.float32),
        pltpu.VMEM_SHARED((NPAD,), jnp.float32),
        pltpu.SemaphoreType.DMA,
    ],
    compiler_params=_sc_params,
)
def _sc_degree(dstm_hbm, out_hbm, dstv, ones_v, zrow_v, acc_sh, sem):
    c = lax.axis_index("c")
    s = lax.axis_index("s")
    w = s * _NC + c
    one16 = jnp.ones((16,), jnp.float32)
    zero16 = jnp.zeros((16,), jnp.float32)
    for i in range(_ECH // 16):
        ones_v[pl.ds(i * 16, 16)] = one16

    def _z(i, _):
        zrow_v[pl.ds(i * 16, 16)] = zero16
        return 0

    lax.fori_loop(0, _RPS // 16, _z, 0)
    pltpu.sync_copy(zrow_v, acc_sh.at[pl.ds(s * _RPS, _RPS)])
    pltpu.sync_copy(dstm_hbm.at[pl.ds(w * _RPW, _RPW)], dstv)
    plsc.subcore_barrier()

    def _body(i, _):
        for j in range(8):
            pltpu.async_copy(ones_v, acc_sh.at[dstv.at[8 * i + j]], sem,
                             add=True)
        for j in range(8):
            pltpu.make_async_copy(ones_v, acc_sh.at[dstv.at[8 * i + j]],
                                  sem).wait()
        return 0

    lax.fori_loop(0, _RPW // 8, _body, 0)
    plsc.subcore_barrier()
    pltpu.sync_copy(acc_sh.at[pl.ds(s * _RPS, _RPS)],
                    out_hbm.at[c, pl.ds(s * _RPS, _RPS)])


def _make_spmm(F):
    @functools.partial(
        pl.kernel,
        out_type=jax.ShapeDtypeStruct((_NC, NPAD, F), jnp.float32),
        mesh=_mesh,
        scratch_types=[
            pltpu.VMEM((_RPW, _ECH), jnp.int32),
            pltpu.VMEM((_RPW, _ECH), jnp.int32),
            pltpu.VMEM((_ECH, F), jnp.float32),
            pltpu.VMEM((_ECH, F), jnp.float32),
            pltpu.VMEM_SHARED((NPAD, F), jnp.float32),
            pltpu.SemaphoreType.DMA,
            pltpu.SemaphoreType.DMA,
        ],
        compiler_params=_sc_params,
    )
    def _spmm(srcm_hbm, dstm_hbm, hs_hbm, out_hbm, srcv, dstv,
              buf_a, buf_b, acc_sh, sem_a, sem_b):
        c = lax.axis_index("c")
        s = lax.axis_index("s")
        w = s * _NC + c
        zero16 = jnp.zeros((16,), jnp.float32)

        def _z(r, _):
            for j in range(F // 16):
                buf_a[r, pl.ds(j * 16, 16)] = zero16
            return 0

        lax.fori_loop(0, _ECH, _z, 0)
        for t in range(_RPS // _ECH):
            pltpu.sync_copy(buf_a, acc_sh.at[pl.ds(s * _RPS + t * _ECH, _ECH)])
        pltpu.sync_copy(srcm_hbm.at[pl.ds(w * _RPW, _RPW)], srcv)
        pltpu.sync_copy(dstm_hbm.at[pl.ds(w * _RPW, _RPW)], dstv)
        plsc.subcore_barrier()

        pltpu.async_copy(hs_hbm.at[srcv.at[0]], buf_a, sem_a)

        def _body(i, _):
            k = 2 * i
            pltpu.async_copy(hs_hbm.at[srcv.at[k + 1]], buf_b, sem_b)
            pltpu.make_async_copy(hs_hbm.at[srcv.at[k]], buf_a, sem_a).wait()
            pltpu.sync_copy(buf_a, acc_sh.at[dstv.at[k]], add=True)

            @pl.when(k + 2 < _RPW)
            def _():
                pltpu.async_copy(hs_hbm.at[srcv.at[k + 2]], buf_a, sem_a)

            pltpu.make_async_copy(hs_hbm.at[srcv.at[k + 1]], buf_b,
                                  sem_b).wait()
            pltpu.sync_copy(buf_b, acc_sh.at[dstv.at[k + 1]], add=True)
            return 0

        lax.fori_loop(0, _RPW // 2, _body, 0)
        plsc.subcore_barrier()
        pltpu.sync_copy(acc_sh.at[pl.ds(s * _RPS, _RPS)],
                        out_hbm.at[c, pl.ds(s * _RPS, _RPS)])

    return _spmm


_spmm32 = _make_spmm(32)
_spmm64 = _make_spmm(64)

_FH = 64
_RPC = _EROWS // _NS


@functools.partial(
    pl.kernel,
    out_type=jax.ShapeDtypeStruct((_NC, NPAD, _FH), jnp.float32),
    mesh=_mesh,
    scratch_types=[
        pltpu.VMEM((_RPC, _ECH), jnp.int32),
        pltpu.VMEM((_RPC, _ECH), jnp.int32),
        pltpu.VMEM((_ECH, _FH), jnp.float32),
        pltpu.VMEM((_ECH, _FH), jnp.float32),
        pltpu.VMEM_SHARED((NPAD, _FH), jnp.float32),
        pltpu.SemaphoreType.DMA,
        pltpu.SemaphoreType.DMA,
    ],
    compiler_params=_sc_params,
)
def _spmm128(srcm_hbm, dstm_hbm, hsa_hbm, hsb_hbm, out_hbm, srcv, dstv,
             buf_a, buf_b, acc_sh, sem_a, sem_b):
    c = lax.axis_index("c")
    s = lax.axis_index("s")
    zero16 = jnp.zeros((16,), jnp.float32)

    def _z(r, _):
        for j in range(_FH // 16):
            buf_a[r, pl.ds(j * 16, 16)] = zero16
        return 0

    lax.fori_loop(0, _ECH, _z, 0)
    for t in range(_RPS // _ECH):
        pltpu.sync_copy(buf_a, acc_sh.at[pl.ds(s * _RPS + t * _ECH, _ECH)])
    pltpu.sync_copy(srcm_hbm.at[pl.ds(s * _RPC, _RPC)], srcv)
    pltpu.sync_copy(dstm_hbm.at[pl.ds(s * _RPC, _RPC)], dstv)
    plsc.subcore_barrier()

    def _run(hs_hbm):
        pltpu.async_copy(hs_hbm.at[srcv.at[0]], buf_a, sem_a)

        def _body(i, _):
            k = 2 * i
            pltpu.async_copy(hs_hbm.at[srcv.at[k + 1]], buf_b, sem_b)
            pltpu.make_async_copy(hs_hbm.at[srcv.at[k]], buf_a, sem_a).wait()
            pltpu.sync_copy(buf_a, acc_sh.at[dstv.at[k]], add=True)

            @pl.when(k + 2 < _RPC)
            def _():
                pltpu.async_copy(hs_hbm.at[srcv.at[k + 2]], buf_a, sem_a)

            pltpu.make_async_copy(hs_hbm.at[srcv.at[k + 1]], buf_b,
                                  sem_b).wait()
            pltpu.sync_copy(buf_b, acc_sh.at[dstv.at[k + 1]], add=True)
            return 0

        lax.fori_loop(0, _RPC // 2, _body, 0)

    @pl.when(c == 0)
    def _():
        _run(hsa_hbm)

    @pl.when(c == 1)
    def _():
        _run(hsb_hbm)

    plsc.subcore_barrier()
    pltpu.sync_copy(acc_sh.at[pl.ds(s * _RPS, _RPS)],
                    out_hbm.at[c, pl.ds(s * _RPS, _RPS)])


def _tc_first(xp, w1, degp):
    def body(x_ref, w1_ref, degp_ref, dinv_ref, hs_ref):
        deg = degp_ref[0] + degp_ref[1] + 1.0
        dinv = lax.rsqrt(deg)
        dinv_ref[...] = dinv
        hs_ref[...] = dinv * jnp.dot(x_ref[...], w1_ref[...],
                                     preferred_element_type=jnp.float32)

    return pl.pallas_call(
        body,
        out_shape=(
            jax.ShapeDtypeStruct((NPAD, 1), jnp.float32),
            jax.ShapeDtypeStruct((NPAD, w1.shape[1]), jnp.float32),
        ),
    )(xp, w1, degp)


def _tc_mid(accp, hs, dinv, b2d, w):
    def body(accp_ref, hs_ref, dinv_ref, b_ref, w_ref, out_ref):
        a = accp_ref[0] + accp_ref[1] + hs_ref[...]
        h = dinv_ref[...] * a + b_ref[...]
        h = jnp.maximum(h, 0.0)
        out_ref[...] = dinv_ref[...] * jnp.dot(h, w_ref[...],
                                               preferred_element_type=jnp.float32)

    return pl.pallas_call(
        body,
        out_shape=jax.ShapeDtypeStruct((NPAD, w.shape[1]), jnp.float32),
    )(accp, hs, dinv, b2d, w)


_RB = 1024
_GRID = NPAD // _RB


def _tc_head(accp, hs3, dinv, b3_2d, batch2d, wl1, bl1, wl2, bl2, wl3, bl3):
    def body(accp_ref, hs_ref, dinv_ref, b3_ref, batch_ref, wl1_ref, bl1_ref,
             wl2_ref, bl2_ref, wl3_ref, bl3_ref, out_ref, sums, cnts):
        i = pl.program_id(0)
        acc = jnp.concatenate([accp_ref[0], accp_ref[1]], axis=1)
        h3 = dinv_ref[...] * (acc + hs_ref[...]) + b3_ref[...]
        z = jnp.maximum(jnp.dot(h3, wl1_ref[...],
                                preferred_element_type=jnp.float32) + bl1_ref[...], 0.0)
        z = jnp.maximum(jnp.dot(z, wl2_ref[...],
                                preferred_element_type=jnp.float32) + bl2_ref[...], 0.0)
        gids = lax.broadcasted_iota(jnp.int32, (NG, _RB), 0)
        onehot = (gids == jnp.broadcast_to(batch_ref[...], (NG, _RB))).astype(jnp.float32)
        psum = jnp.dot(onehot, z, preferred_element_type=jnp.float32)
        pcnt = jnp.sum(onehot, axis=1, keepdims=True)

        @pl.when(i == 0)
        def _():
            sums[...] = jnp.zeros_like(sums)
            cnts[...] = jnp.zeros_like(cnts)

        sums[...] += psum
        cnts[...] += pcnt

        @pl.when(i == _GRID - 1)
        def _():
            mean = sums[...] / jnp.maximum(cnts[...], 1.0)
            out_ref[...] = jnp.dot(mean, wl3_ref[...],
                                   preferred_element_type=jnp.float32) + bl3_ref[...]

    h2 = wl2.shape[0]
    h3w = wl2.shape[1]
    return pl.pallas_call(
        body,
        grid=(_GRID,),
        in_specs=[
            pl.BlockSpec((_NC, _RB, _FH), lambda i: (0, i, 0)),
            pl.BlockSpec((_RB, 128), lambda i: (i, 0)),
            pl.BlockSpec((_RB, 1), lambda i: (i, 0)),
            pl.BlockSpec((1, 128), lambda i: (0, 0)),
            pl.BlockSpec((1, _RB), lambda i: (0, i)),
            pl.BlockSpec((128, h2), lambda i: (0, 0)),
            pl.BlockSpec((1, h2), lambda i: (0, 0)),
            pl.BlockSpec((h2, h3w), lambda i: (0, 0)),
            pl.BlockSpec((1, h3w), lambda i: (0, 0)),
            pl.BlockSpec((h3w, 4), lambda i: (0, 0)),
            pl.BlockSpec((1, 4), lambda i: (0, 0)),
        ],
        out_specs=pl.BlockSpec((NG, 4), lambda i: (0, 0)),
        out_shape=jax.ShapeDtypeStruct((NG, 4), jnp.float32),
        scratch_shapes=[
            pltpu.VMEM((NG, h3w), jnp.float32),
            pltpu.VMEM((NG, 1), jnp.float32),
        ],
    )(accp, hs3, dinv, b3_2d, batch2d, wl1, bl1, wl2, bl2, wl3, bl3)


def kernel(x, edge_index, batch, W1, b1, W2, b2, W3, b3,
           Wl1, bl1, Wl2, bl2, Wl3, bl3):
    epad = jnp.full((2, _EPAD - E), NPAD - 1, jnp.int32)
    em = jnp.concatenate([edge_index.astype(jnp.int32), epad], axis=1)
    srcm = em[0].reshape(_EROWS, _ECH)
    dstm = em[1].reshape(_EROWS, _ECH)
    xp = jnp.pad(x, ((0, NPAD - N), (0, 0)))
    batch2d = jnp.pad(batch.astype(jnp.int32), (0, NPAD - N),
                      constant_values=NG).reshape(1, NPAD)

    degp = _sc_degree(dstm).reshape(_NC, NPAD, 1)
    dinv, hs1 = _tc_first(xp, W1, degp)
    acc1 = _spmm32(srcm, dstm, hs1)
    hs2 = _tc_mid(acc1, hs1, dinv, b1.reshape(1, -1), W2)
    acc2 = _spmm64(srcm, dstm, hs2)
    hs3 = _tc_mid(acc2, hs2, dinv, b2.reshape(1, -1), W3)
    acc3 = _spmm128(srcm, dstm, hs3[:, :_FH], hs3[:, _FH:])
    return _tc_head(acc3, hs3, dinv, b3.reshape(1, -1), batch2d,
                    Wl1, bl1.reshape(1, -1), Wl2, bl2.reshape(1, -1),
                    Wl3, bl3.reshape(1, -1))


# speedup vs baseline: 14.9383x; 1.0248x over previous
import functools

import jax
import jax.numpy as jnp
from jax import lax
from jax.experimental import pallas as pl
from jax.experimental.pallas import tpu as pltpu
from jax.experimental.pallas import tpu_sc as plsc

N = 10000
NPAD = 10240
E = 320000
NG = 64

_NC = 2
_NS = 16
_NW = _NC * _NS
_ECH = 128
_EROWS = 2560
_EPAD = _ECH * _EROWS
_RPW = _EROWS // _NW
_RPS = NPAD // _NS

_mesh = plsc.VectorSubcoreMesh(core_axis_name="c", subcore_axis_name="s")
_sc_params = pltpu.CompilerParams(use_tc_tiling_on_sc=False)

_NB = 8


def _zero_fill(buf, rows, width):
    zero16 = jnp.zeros((16,), jnp.float32)

    def _z(r, _):
        for j in range(width // 16):
            buf[r, pl.ds(j * 16, 16)] = zero16
        return 0

    lax.fori_loop(0, rows, _z, 0)


def _edge_pipeline(hs_hbm, srcv, dstv, bufs, gsems, ssem, acc_sh, nrows):
    for j in range(_NB):
        pltpu.async_copy(hs_hbm.at[srcv.at[j]], bufs[j], gsems[j])

    def _grp(i, _):
        k0 = _NB * i
        for j in range(_NB):
            pltpu.make_async_copy(hs_hbm.at[srcv.at[k0 + j]], bufs[j],
                                  gsems[j]).wait()
            pltpu.async_copy(bufs[j], acc_sh.at[dstv.at[k0 + j]], ssem,
                             add=True)
        for j in range(_NB):
            pltpu.make_async_copy(bufs[j], acc_sh.at[dstv.at[k0 + j]],
                                  ssem).wait()
        for j in range(_NB):
            nxt = k0 + _NB + j

            @pl.when(nxt < nrows)
            def _():
                pltpu.async_copy(hs_hbm.at[srcv.at[nxt]], bufs[j], gsems[j])

        return 0

    lax.fori_loop(0, nrows // _NB, _grp, 0)


@functools.partial(
    pl.kernel,
    out_type=jax.ShapeDtypeStruct((_NC, NPAD), jnp.float32),
    mesh=_mesh,
    scratch_types=[
        pltpu.VMEM((_RPW, _ECH), jnp.int32),
        pltpu.VMEM((_ECH,), jnp.float32),
        pltpu.VMEM((_RPS,), jnp.float32),
        pltpu.VMEM_SHARED((NPAD,), jnp.float32),
        pltpu.SemaphoreType.DMA,
    ],
    compiler_params=_sc_params,
)
def _sc_degree(dstm_hbm, out_hbm, dstv, ones_v, zrow_v, acc_sh, sem):
    c = lax.axis_index("c")
    s = lax.axis_index("s")
    w = s * _NC + c
    one16 = jnp.ones((16,), jnp.float32)
    zero16 = jnp.zeros((16,), jnp.float32)
    for i in range(_ECH // 16):
        ones_v[pl.ds(i * 16, 16)] = one16

    def _z(i, _):
        zrow_v[pl.ds(i * 16, 16)] = zero16
        return 0

    lax.fori_loop(0, _RPS // 16, _z, 0)
    pltpu.sync_copy(zrow_v, acc_sh.at[pl.ds(s * _RPS, _RPS)])
    pltpu.sync_copy(dstm_hbm.at[pl.ds(w * _RPW, _RPW)], dstv)
    plsc.subcore_barrier()

    def _body(i, _):
        for j in range(8):
            pltpu.async_copy(ones_v, acc_sh.at[dstv.at[8 * i + j]], sem,
                             add=True)
        for j in range(8):
            pltpu.make_async_copy(ones_v, acc_sh.at[dstv.at[8 * i + j]],
                                  sem).wait()
        return 0

    lax.fori_loop(0, _RPW // 8, _body, 0)
    plsc.subcore_barrier()
    pltpu.sync_copy(acc_sh.at[pl.ds(s * _RPS, _RPS)],
                    out_hbm.at[c, pl.ds(s * _RPS, _RPS)])


_RPC = _EROWS // _NS


def _make_spmm(FH):
    @functools.partial(
        pl.kernel,
        out_type=jax.ShapeDtypeStruct((_NC, NPAD, FH), jnp.float32),
        mesh=_mesh,
        scratch_types=[
            pltpu.VMEM((_RPC, _ECH), jnp.int32),
            pltpu.VMEM((_RPC, _ECH), jnp.int32),
            [pltpu.VMEM((_ECH, FH), jnp.float32)] * _NB,
            pltpu.VMEM_SHARED((NPAD, FH), jnp.float32),
            [pltpu.SemaphoreType.DMA] * _NB,
            pltpu.SemaphoreType.DMA,
        ],
        compiler_params=_sc_params,
    )
    def _spmm(srcm_hbm, dstm_hbm, hsa_hbm, hsb_hbm, out_hbm, srcv, dstv,
              bufs, acc_sh, gsems, ssem):
        c = lax.axis_index("c")
        s = lax.axis_index("s")
        _zero_fill(bufs[0], _ECH, FH)
        for t in range(_RPS // _ECH):
            pltpu.sync_copy(bufs[0], acc_sh.at[pl.ds(s * _RPS + t * _ECH, _ECH)])
        pltpu.sync_copy(srcm_hbm.at[pl.ds(s * _RPC, _RPC)], srcv)
        pltpu.sync_copy(dstm_hbm.at[pl.ds(s * _RPC, _RPC)], dstv)
        plsc.subcore_barrier()

        @pl.when(c == 0)
        def _():
            _edge_pipeline(hsa_hbm, srcv, dstv, bufs, gsems, ssem, acc_sh, _RPC)

        @pl.when(c == 1)
        def _():
            _edge_pipeline(hsb_hbm, srcv, dstv, bufs, gsems, ssem, acc_sh, _RPC)

        plsc.subcore_barrier()
        pltpu.sync_copy(acc_sh.at[pl.ds(s * _RPS, _RPS)],
                        out_hbm.at[c, pl.ds(s * _RPS, _RPS)])

    return _spmm


_spmm32 = _make_spmm(16)
_spmm64 = _make_spmm(32)

_FQ = 32


@functools.partial(
    pl.kernel,
    out_type=jax.ShapeDtypeStruct((4, NPAD, _FQ), jnp.float32),
    mesh=_mesh,
    scratch_types=[
        pltpu.VMEM((_RPC, _ECH), jnp.int32),
        pltpu.VMEM((_RPC, _ECH), jnp.int32),
        [pltpu.VMEM((_ECH, _FQ), jnp.float32)] * _NB,
        pltpu.VMEM_SHARED((NPAD, _FQ), jnp.float32),
        [pltpu.SemaphoreType.DMA] * _NB,
        pltpu.SemaphoreType.DMA,
    ],
    compiler_params=_sc_params,
)
def _spmm128(srcm_hbm, dstm_hbm, hq0_hbm, hq1_hbm, hq2_hbm, hq3_hbm,
             out_hbm, srcv, dstv, bufs, acc_sh, gsems, ssem):
    c = lax.axis_index("c")
    s = lax.axis_index("s")
    pltpu.sync_copy(srcm_hbm.at[pl.ds(s * _RPC, _RPC)], srcv)
    pltpu.sync_copy(dstm_hbm.at[pl.ds(s * _RPC, _RPC)], dstv)
    quarters = ((hq0_hbm, hq2_hbm), (hq1_hbm, hq3_hbm))
    for q in range(2):
        _zero_fill(bufs[0], _ECH, _FQ)
        for t in range(_RPS // _ECH):
            pltpu.sync_copy(bufs[0],
                            acc_sh.at[pl.ds(s * _RPS + t * _ECH, _ECH)])
        plsc.subcore_barrier()

        @pl.when(c == 0)
        def _():
            _edge_pipeline(quarters[q][0], srcv, dstv, bufs, gsems, ssem,
                           acc_sh, _RPC)

        @pl.when(c == 1)
        def _():
            _edge_pipeline(quarters[q][1], srcv, dstv, bufs, gsems, ssem,
                           acc_sh, _RPC)

        plsc.subcore_barrier()
        pltpu.sync_copy(acc_sh.at[pl.ds(s * _RPS, _RPS)],
                        out_hbm.at[c * 2 + q, pl.ds(s * _RPS, _RPS)])
        plsc.subcore_barrier()


def _tc_first(xp, w1, degp):
    def body(x_ref, w1_ref, degp_ref, dinv_ref, hs_ref):
        deg = degp_ref[0] + degp_ref[1] + 1.0
        dinv = lax.rsqrt(deg)
        dinv_ref[...] = dinv
        hs_ref[...] = dinv * jnp.dot(x_ref[...], w1_ref[...],
                                     preferred_element_type=jnp.float32)

    return pl.pallas_call(
        body,
        out_shape=(
            jax.ShapeDtypeStruct((NPAD, 1), jnp.float32),
            jax.ShapeDtypeStruct((NPAD, w1.shape[1]), jnp.float32),
        ),
    )(xp, w1, degp)


def _tc_mid(accp, hs, dinv, b2d, w):
    def body(accp_ref, hs_ref, dinv_ref, b_ref, w_ref, out_ref):
        acc = jnp.concatenate([accp_ref[i] for i in range(accp.shape[0])],
                              axis=1)
        h = dinv_ref[...] * (acc + hs_ref[...]) + b_ref[...]
        h = jnp.maximum(h, 0.0)
        out_ref[...] = dinv_ref[...] * jnp.dot(h, w_ref[...],
                                               preferred_element_type=jnp.float32)

    return pl.pallas_call(
        body,
        out_shape=jax.ShapeDtypeStruct((NPAD, w.shape[1]), jnp.float32),
    )(accp, hs, dinv, b2d, w)


_RB = 1024
_GRID = NPAD // _RB


def _tc_head(accp, hs3, dinv, b3_2d, batch2d, wl1, bl1, wl2, bl2, wl3, bl3):
    def body(accp_ref, hs_ref, dinv_ref, b3_ref, batch_ref, wl1_ref, bl1_ref,
             wl2_ref, bl2_ref, wl3_ref, bl3_ref, out_ref, sums, cnts):
        i = pl.program_id(0)
        acc = jnp.concatenate([accp_ref[q] for q in range(4)], axis=1)
        h3 = dinv_ref[...] * (acc + hs_ref[...]) + b3_ref[...]
        z = jnp.maximum(jnp.dot(h3, wl1_ref[...],
                                preferred_element_type=jnp.float32) + bl1_ref[...], 0.0)
        z = jnp.maximum(jnp.dot(z, wl2_ref[...],
                                preferred_element_type=jnp.float32) + bl2_ref[...], 0.0)
        gids = lax.broadcasted_iota(jnp.int32, (NG, _RB), 0)
        onehot = (gids == jnp.broadcast_to(batch_ref[...], (NG, _RB))).astype(jnp.float32)
        psum = jnp.dot(onehot, z, preferred_element_type=jnp.float32)
        pcnt = jnp.sum(onehot, axis=1, keepdims=True)

        @pl.when(i == 0)
        def _():
            sums[...] = jnp.zeros_like(sums)
            cnts[...] = jnp.zeros_like(cnts)

        sums[...] += psum
        cnts[...] += pcnt

        @pl.when(i == _GRID - 1)
        def _():
            mean = sums[...] / jnp.maximum(cnts[...], 1.0)
            out_ref[...] = jnp.dot(mean, wl3_ref[...],
                                   preferred_element_type=jnp.float32) + bl3_ref[...]

    h2 = wl2.shape[0]
    h3w = wl2.shape[1]
    return pl.pallas_call(
        body,
        grid=(_GRID,),
        in_specs=[
            pl.BlockSpec((4, _RB, _FQ), lambda i: (0, i, 0)),
            pl.BlockSpec((_RB, 128), lambda i: (i, 0)),
            pl.BlockSpec((_RB, 1), lambda i: (i, 0)),
            pl.BlockSpec((1, 128), lambda i: (0, 0)),
            pl.BlockSpec((1, _RB), lambda i: (0, i)),
            pl.BlockSpec((128, h2), lambda i: (0, 0)),
            pl.BlockSpec((1, h2), lambda i: (0, 0)),
            pl.BlockSpec((h2, h3w), lambda i: (0, 0)),
            pl.BlockSpec((1, h3w), lambda i: (0, 0)),
            pl.BlockSpec((h3w, 4), lambda i: (0, 0)),
            pl.BlockSpec((1, 4), lambda i: (0, 0)),
        ],
        out_specs=pl.BlockSpec((NG, 4), lambda i: (0, 0)),
        out_shape=jax.ShapeDtypeStruct((NG, 4), jnp.float32),
        scratch_shapes=[
            pltpu.VMEM((NG, h3w), jnp.float32),
            pltpu.VMEM((NG, 1), jnp.float32),
        ],
    )(accp, hs3, dinv, b3_2d, batch2d, wl1, bl1, wl2, bl2, wl3, bl3)


def kernel(x, edge_index, batch, W1, b1, W2, b2, W3, b3,
           Wl1, bl1, Wl2, bl2, Wl3, bl3):
    epad = jnp.full((2, _EPAD - E), NPAD - 1, jnp.int32)
    em = jnp.concatenate([edge_index.astype(jnp.int32), epad], axis=1)
    srcm = em[0].reshape(_EROWS, _ECH)
    dstm = em[1].reshape(_EROWS, _ECH)
    xp = jnp.pad(x, ((0, NPAD - N), (0, 0)))
    batch2d = jnp.pad(batch.astype(jnp.int32), (0, NPAD - N),
                      constant_values=NG).reshape(1, NPAD)

    degp = _sc_degree(dstm).reshape(_NC, NPAD, 1)
    dinv, hs1 = _tc_first(xp, W1, degp)
    acc1 = _spmm32(srcm, dstm, hs1[:, :16], hs1[:, 16:])
    hs2 = _tc_mid(acc1, hs1, dinv, b1.reshape(1, -1), W2)
    acc2 = _spmm64(srcm, dstm, hs2[:, :32], hs2[:, 32:])
    hs3 = _tc_mid(acc2, hs2, dinv, b2.reshape(1, -1), W3)
    acc3 = _spmm128(srcm, dstm, hs3[:, :32], hs3[:, 32:64],
                    hs3[:, 64:96], hs3[:, 96:])
    return _tc_head(acc3, hs3, dinv, b3.reshape(1, -1), batch2d,
                    Wl1, bl1.reshape(1, -1), Wl2, bl2.reshape(1, -1),
                    Wl3, bl3.reshape(1, -1))


# speedup vs baseline: 21.9385x; 1.4686x over previous
import functools

import jax
import jax.numpy as jnp
from jax import lax
from jax.experimental import pallas as pl
from jax.experimental.pallas import tpu as pltpu
from jax.experimental.pallas import tpu_sc as plsc

N = 10000
NPAD = 10240
E = 320000
NG = 64

_NC = 2
_NS = 16
_NW = _NC * _NS
_ECH = 128
_EROWS = 2560
_EPAD = _ECH * _EROWS
_RPW = _EROWS // _NW
_RPS = NPAD // _NS

_mesh = plsc.VectorSubcoreMesh(core_axis_name="c", subcore_axis_name="s")
_sc_params = pltpu.CompilerParams(use_tc_tiling_on_sc=False)

_NB = 8


def _zero_fill(buf, rows, width):
    zero16 = jnp.zeros((16,), jnp.float32)

    def _z(r, _):
        for j in range(width // 16):
            buf[r, pl.ds(j * 16, 16)] = zero16
        return 0

    lax.fori_loop(0, rows, _z, 0)


def _edge_pipeline(hs_hbm, srcv, dstv, bufs, gsems, ssem, acc_sh, nrows):
    for j in range(_NB):
        pltpu.async_copy(hs_hbm.at[srcv.at[j]], bufs[j], gsems[j])

    def _grp(i, _):
        k0 = _NB * i
        for j in range(_NB):
            pltpu.make_async_copy(hs_hbm.at[srcv.at[k0 + j]], bufs[j],
                                  gsems[j]).wait()
            pltpu.async_copy(bufs[j], acc_sh.at[dstv.at[k0 + j]], ssem,
                             add=True)
        for j in range(_NB):
            pltpu.make_async_copy(bufs[j], acc_sh.at[dstv.at[k0 + j]],
                                  ssem).wait()
        for j in range(_NB):
            nxt = k0 + _NB + j

            @pl.when(nxt < nrows)
            def _():
                pltpu.async_copy(hs_hbm.at[srcv.at[nxt]], bufs[j], gsems[j])

        return 0

    lax.fori_loop(0, nrows // _NB, _grp, 0)


@functools.partial(
    pl.kernel,
    out_type=jax.ShapeDtypeStruct((_NC, NPAD), jnp.float32),
    mesh=_mesh,
    scratch_types=[
        pltpu.VMEM((_RPW, _ECH), jnp.int32),
        pltpu.VMEM((_ECH,), jnp.float32),
        pltpu.VMEM((_RPS,), jnp.float32),
        pltpu.VMEM_SHARED((NPAD,), jnp.float32),
        pltpu.SemaphoreType.DMA,
    ],
    compiler_params=_sc_params,
)
def _sc_degree(dstm_hbm, out_hbm, dstv, ones_v, zrow_v, acc_sh, sem):
    c = lax.axis_index("c")
    s = lax.axis_index("s")
    w = s * _NC + c
    one16 = jnp.ones((16,), jnp.float32)
    zero16 = jnp.zeros((16,), jnp.float32)
    for i in range(_ECH // 16):
        ones_v[pl.ds(i * 16, 16)] = one16

    def _z(i, _):
        zrow_v[pl.ds(i * 16, 16)] = zero16
        return 0

    lax.fori_loop(0, _RPS // 16, _z, 0)
    pltpu.sync_copy(zrow_v, acc_sh.at[pl.ds(s * _RPS, _RPS)])
    pltpu.sync_copy(dstm_hbm.at[pl.ds(w * _RPW, _RPW)], dstv)
    plsc.subcore_barrier()

    def _body(i, _):
        for j in range(8):
            pltpu.async_copy(ones_v, acc_sh.at[dstv.at[8 * i + j]], sem,
                             add=True)
        for j in range(8):
            pltpu.make_async_copy(ones_v, acc_sh.at[dstv.at[8 * i + j]],
                                  sem).wait()
        return 0

    lax.fori_loop(0, _RPW // 8, _body, 0)
    plsc.subcore_barrier()
    pltpu.sync_copy(acc_sh.at[pl.ds(s * _RPS, _RPS)],
                    out_hbm.at[c, pl.ds(s * _RPS, _RPS)])


_RPC = _EROWS // _NS


def _make_spmm(FH):
    @functools.partial(
        pl.kernel,
        out_type=jax.ShapeDtypeStruct((_NC, NPAD, FH), jnp.float32),
        mesh=_mesh,
        scratch_types=[
            pltpu.VMEM((_RPC, _ECH), jnp.int32),
            pltpu.VMEM((_RPC, _ECH), jnp.int32),
            [pltpu.VMEM((_ECH, FH), jnp.float32)] * _NB,
            pltpu.VMEM_SHARED((NPAD, FH), jnp.float32),
            [pltpu.SemaphoreType.DMA] * _NB,
            pltpu.SemaphoreType.DMA,
        ],
        compiler_params=_sc_params,
    )
    def _spmm(srcm_hbm, dstm_hbm, hsa_hbm, hsb_hbm, out_hbm, srcv, dstv,
              bufs, acc_sh, gsems, ssem):
        c = lax.axis_index("c")
        s = lax.axis_index("s")
        _zero_fill(bufs[0], _ECH, FH)
        for t in range(_RPS // _ECH):
            pltpu.sync_copy(bufs[0], acc_sh.at[pl.ds(s * _RPS + t * _ECH, _ECH)])
        pltpu.sync_copy(srcm_hbm.at[pl.ds(s * _RPC, _RPC)], srcv)
        pltpu.sync_copy(dstm_hbm.at[pl.ds(s * _RPC, _RPC)], dstv)
        plsc.subcore_barrier()

        @pl.when(c == 0)
        def _():
            _edge_pipeline(hsa_hbm, srcv, dstv, bufs, gsems, ssem, acc_sh, _RPC)

        @pl.when(c == 1)
        def _():
            _edge_pipeline(hsb_hbm, srcv, dstv, bufs, gsems, ssem, acc_sh, _RPC)

        plsc.subcore_barrier()
        pltpu.sync_copy(acc_sh.at[pl.ds(s * _RPS, _RPS)],
                        out_hbm.at[c, pl.ds(s * _RPS, _RPS)])

    return _spmm


_spmm16 = _make_spmm(16)
_spmm32 = _make_spmm(32)


def _tc_first(xp, w1, degp):
    def body(x_ref, w1_ref, degp_ref, dinv_ref, hs_ref):
        deg = degp_ref[0] + degp_ref[1] + 1.0
        dinv = lax.rsqrt(deg)
        dinv_ref[...] = dinv
        hs_ref[...] = dinv * jnp.dot(x_ref[...], w1_ref[...],
                                     preferred_element_type=jnp.float32)

    return pl.pallas_call(
        body,
        out_shape=(
            jax.ShapeDtypeStruct((NPAD, 1), jnp.float32),
            jax.ShapeDtypeStruct((NPAD, w1.shape[1]), jnp.float32),
        ),
    )(xp, w1, degp)


def _tc_elem(accp, hs, dinv, b2d):
    def body(accp_ref, hs_ref, dinv_ref, b_ref, out_ref):
        acc = jnp.concatenate([accp_ref[0], accp_ref[1]], axis=1)
        h = jnp.maximum(dinv_ref[...] * (acc + hs_ref[...]) + b_ref[...], 0.0)
        out_ref[...] = dinv_ref[...] * h

    return pl.pallas_call(
        body,
        out_shape=jax.ShapeDtypeStruct((NPAD, hs.shape[1]), jnp.float32),
    )(accp, hs, dinv, b2d)


def _tc_matmul(accp, p, dinv, b2d, w):
    def body(accp_ref, p_ref, dinv_ref, b_ref, w_ref, out_ref):
        acc = jnp.concatenate([accp_ref[0], accp_ref[1]], axis=1)
        t = dinv_ref[...] * (acc + p_ref[...])
        h = jnp.maximum(jnp.dot(t, w_ref[...],
                                preferred_element_type=jnp.float32) + b_ref[...],
                        0.0)
        out_ref[...] = dinv_ref[...] * h

    return pl.pallas_call(
        body,
        out_shape=jax.ShapeDtypeStruct((NPAD, w.shape[1]), jnp.float32),
    )(accp, p, dinv, b2d, w)


_RB = 1024
_GRID = NPAD // _RB


def _tc_head(accp, p3, dinv, w3, b3_2d, batch2d, wl1, bl1, wl2, bl2, wl3, bl3):
    def body(accp_ref, p_ref, dinv_ref, w3_ref, b3_ref, batch_ref, wl1_ref,
             bl1_ref, wl2_ref, bl2_ref, wl3_ref, bl3_ref, out_ref, sums, cnts):
        i = pl.program_id(0)
        acc = jnp.concatenate([accp_ref[0], accp_ref[1]], axis=1)
        t3 = dinv_ref[...] * (acc + p_ref[...])
        h3 = jnp.dot(t3, w3_ref[...],
                     preferred_element_type=jnp.float32) + b3_ref[...]
        z = jnp.maximum(jnp.dot(h3, wl1_ref[...],
                                preferred_element_type=jnp.float32) + bl1_ref[...], 0.0)
        z = jnp.maximum(jnp.dot(z, wl2_ref[...],
                                preferred_element_type=jnp.float32) + bl2_ref[...], 0.0)
        gids = lax.broadcasted_iota(jnp.int32, (NG, _RB), 0)
        onehot = (gids == jnp.broadcast_to(batch_ref[...], (NG, _RB))).astype(jnp.float32)
        psum = jnp.dot(onehot, z, preferred_element_type=jnp.float32)
        pcnt = jnp.sum(onehot, axis=1, keepdims=True)

        @pl.when(i == 0)
        def _():
            sums[...] = jnp.zeros_like(sums)
            cnts[...] = jnp.zeros_like(cnts)

        sums[...] += psum
        cnts[...] += pcnt

        @pl.when(i == _GRID - 1)
        def _():
            mean = sums[...] / jnp.maximum(cnts[...], 1.0)
            out_ref[...] = jnp.dot(mean, wl3_ref[...],
                                   preferred_element_type=jnp.float32) + bl3_ref[...]

    h2 = wl2.shape[0]
    h3w = wl2.shape[1]
    return pl.pallas_call(
        body,
        grid=(_GRID,),
        in_specs=[
            pl.BlockSpec((2, _RB, 32), lambda i: (0, i, 0)),
            pl.BlockSpec((_RB, 64), lambda i: (i, 0)),
            pl.BlockSpec((_RB, 1), lambda i: (i, 0)),
            pl.BlockSpec((64, 128), lambda i: (0, 0)),
            pl.BlockSpec((1, 128), lambda i: (0, 0)),
            pl.BlockSpec((1, _RB), lambda i: (0, i)),
            pl.BlockSpec((128, h2), lambda i: (0, 0)),
            pl.BlockSpec((1, h2), lambda i: (0, 0)),
            pl.BlockSpec((h2, h3w), lambda i: (0, 0)),
            pl.BlockSpec((1, h3w), lambda i: (0, 0)),
            pl.BlockSpec((h3w, 4), lambda i: (0, 0)),
            pl.BlockSpec((1, 4), lambda i: (0, 0)),
        ],
        out_specs=pl.BlockSpec((NG, 4), lambda i: (0, 0)),
        out_shape=jax.ShapeDtypeStruct((NG, 4), jnp.float32),
        scratch_shapes=[
            pltpu.VMEM((NG, h3w), jnp.float32),
            pltpu.VMEM((NG, 1), jnp.float32),
        ],
    )(accp, p3, dinv, w3, b3_2d, batch2d, wl1, bl1, wl2, bl2, wl3, bl3)


def kernel(x, edge_index, batch, W1, b1, W2, b2, W3, b3,
           Wl1, bl1, Wl2, bl2, Wl3, bl3):
    epad = jnp.full((2, _EPAD - E), NPAD - 1, jnp.int32)
    em = jnp.concatenate([edge_index.astype(jnp.int32), epad], axis=1)
    srcm = em[0].reshape(_EROWS, _ECH)
    dstm = em[1].reshape(_EROWS, _ECH)
    xp = jnp.pad(x, ((0, NPAD - N), (0, 0)))
    batch2d = jnp.pad(batch.astype(jnp.int32), (0, NPAD - N),
                      constant_values=NG).reshape(1, NPAD)

    degp = _sc_degree(dstm).reshape(_NC, NPAD, 1)
    dinv, hs1 = _tc_first(xp, W1, degp)
    acc1 = _spmm16(srcm, dstm, hs1[:, :16], hs1[:, 16:])
    p2 = _tc_elem(acc1, hs1, dinv, b1.reshape(1, -1))
    acc2 = _spmm16(srcm, dstm, p2[:, :16], p2[:, 16:])
    p3 = _tc_matmul(acc2, p2, dinv, b2.reshape(1, -1), W2)
    acc3 = _spmm32(srcm, dstm, p3[:, :32], p3[:, 32:])
    return _tc_head(acc3, p3, dinv, W3, b3.reshape(1, -1), batch2d,
                    Wl1, bl1.reshape(1, -1), Wl2, bl2.reshape(1, -1),
                    Wl3, bl3.reshape(1, -1))


# speedup vs baseline: 22.3627x; 1.0193x over previous
import functools

import jax
import jax.numpy as jnp
from jax import lax
from jax.experimental import pallas as pl
from jax.experimental.pallas import tpu as pltpu
from jax.experimental.pallas import tpu_sc as plsc

N = 10000
NPAD = 10240
E = 320000
NG = 64

_NC = 2
_NS = 16
_NW = _NC * _NS
_ECH = 128
_EROWS = 2560
_EPAD = _ECH * _EROWS
_RPW = _EROWS // _NW
_RPS = NPAD // _NS

_mesh = plsc.VectorSubcoreMesh(core_axis_name="c", subcore_axis_name="s")
_sc_params = pltpu.CompilerParams(use_tc_tiling_on_sc=False)

_NB = 8


def _zero_fill(buf, rows, width):
    zero16 = jnp.zeros((16,), jnp.float32)

    def _z(r, _):
        for j in range(width // 16):
            buf[r, pl.ds(j * 16, 16)] = zero16
        return 0

    lax.fori_loop(0, rows, _z, 0)


def _edge_pipeline(hs_hbm, srcv, dstv, bufs, gsems, ssem, acc_sh, nrows):
    for j in range(_NB):
        pltpu.async_copy(hs_hbm.at[srcv.at[j]], bufs[j], gsems[j])

    def _grp(i, _):
        k0 = _NB * i
        for j in range(_NB):
            pltpu.make_async_copy(hs_hbm.at[srcv.at[k0 + j]], bufs[j],
                                  gsems[j]).wait()
            pltpu.async_copy(bufs[j], acc_sh.at[dstv.at[k0 + j]], ssem,
                             add=True)
        for j in range(_NB):
            pltpu.make_async_copy(bufs[j], acc_sh.at[dstv.at[k0 + j]],
                                  ssem).wait()
        for j in range(_NB):
            nxt = k0 + _NB + j

            @pl.when(nxt < nrows)
            def _():
                pltpu.async_copy(hs_hbm.at[srcv.at[nxt]], bufs[j], gsems[j])

        return 0

    lax.fori_loop(0, nrows // _NB, _grp, 0)


@functools.partial(
    pl.kernel,
    out_type=jax.ShapeDtypeStruct((_NC, NPAD), jnp.float32),
    mesh=_mesh,
    scratch_types=[
        pltpu.VMEM((_RPW, _ECH), jnp.int32),
        pltpu.VMEM((_ECH,), jnp.float32),
        pltpu.VMEM((_RPS,), jnp.float32),
        pltpu.VMEM_SHARED((NPAD,), jnp.float32),
        pltpu.SemaphoreType.DMA,
    ],
    compiler_params=_sc_params,
)
def _sc_degree(dstm_hbm, out_hbm, dstv, ones_v, zrow_v, acc_sh, sem):
    c = lax.axis_index("c")
    s = lax.axis_index("s")
    w = s * _NC + c
    one16 = jnp.ones((16,), jnp.float32)
    zero16 = jnp.zeros((16,), jnp.float32)
    for i in range(_ECH // 16):
        ones_v[pl.ds(i * 16, 16)] = one16

    def _z(i, _):
        zrow_v[pl.ds(i * 16, 16)] = zero16
        return 0

    lax.fori_loop(0, _RPS // 16, _z, 0)
    pltpu.sync_copy(zrow_v, acc_sh.at[pl.ds(s * _RPS, _RPS)])
    pltpu.sync_copy(dstm_hbm.at[pl.ds(w * _RPW, _RPW)], dstv)
    plsc.subcore_barrier()

    def _body(i, _):
        for j in range(8):
            pltpu.async_copy(ones_v, acc_sh.at[dstv.at[8 * i + j]], sem,
                             add=True)
        for j in range(8):
            pltpu.make_async_copy(ones_v, acc_sh.at[dstv.at[8 * i + j]],
                                  sem).wait()
        return 0

    lax.fori_loop(0, _RPW // 8, _body, 0)
    plsc.subcore_barrier()
    pltpu.sync_copy(acc_sh.at[pl.ds(s * _RPS, _RPS)],
                    out_hbm.at[c, pl.ds(s * _RPS, _RPS)])


_RPC = _EROWS // _NS


def _make_spmm(FH):
    @functools.partial(
        pl.kernel,
        out_type=jax.ShapeDtypeStruct((_NC, NPAD, FH), jnp.float32),
        mesh=_mesh,
        scratch_types=[
            pltpu.VMEM((_RPC, _ECH), jnp.int32),
            pltpu.VMEM((_RPC, _ECH), jnp.int32),
            [pltpu.VMEM((_ECH, FH), jnp.float32)] * _NB,
            pltpu.VMEM_SHARED((NPAD, FH), jnp.float32),
            [pltpu.SemaphoreType.DMA] * _NB,
            pltpu.SemaphoreType.DMA,
        ],
        compiler_params=_sc_params,
    )
    def _spmm(srcm_hbm, dstm_hbm, hsa_hbm, hsb_hbm, out_hbm, srcv, dstv,
              bufs, acc_sh, gsems, ssem):
        c = lax.axis_index("c")
        s = lax.axis_index("s")
        _zero_fill(bufs[0], _ECH, FH)
        for t in range(_RPS // _ECH):
            pltpu.sync_copy(bufs[0], acc_sh.at[pl.ds(s * _RPS + t * _ECH, _ECH)])
        pltpu.sync_copy(srcm_hbm.at[pl.ds(s * _RPC, _RPC)], srcv)
        pltpu.sync_copy(dstm_hbm.at[pl.ds(s * _RPC, _RPC)], dstv)
        plsc.subcore_barrier()

        @pl.when(c == 0)
        def _():
            _edge_pipeline(hsa_hbm, srcv, dstv, bufs, gsems, ssem, acc_sh, _RPC)

        @pl.when(c == 1)
        def _():
            _edge_pipeline(hsb_hbm, srcv, dstv, bufs, gsems, ssem, acc_sh, _RPC)

        plsc.subcore_barrier()
        pltpu.sync_copy(acc_sh.at[pl.ds(s * _RPS, _RPS)],
                        out_hbm.at[c, pl.ds(s * _RPS, _RPS)])

    return _spmm


_spmm16 = _make_spmm(16)
_spmm32 = _make_spmm(32)


def _tc_xw(xp, w1):
    def body(x_ref, w1_ref, out_ref):
        out_ref[...] = jnp.dot(x_ref[...], w1_ref[...],
                               preferred_element_type=jnp.float32)

    return pl.pallas_call(
        body,
        out_shape=jax.ShapeDtypeStruct((NPAD, w1.shape[1]), jnp.float32),
    )(xp, w1)


def _tc_scale(hpre, degp):
    def body(hpre_ref, degp_ref, dinv_ref, hs_ref):
        deg = degp_ref[0] + degp_ref[1] + 1.0
        dinv = lax.rsqrt(deg)
        dinv_ref[...] = dinv
        hs_ref[...] = dinv * hpre_ref[...]

    return pl.pallas_call(
        body,
        out_shape=(
            jax.ShapeDtypeStruct((NPAD, 1), jnp.float32),
            jax.ShapeDtypeStruct((NPAD, hpre.shape[1]), jnp.float32),
        ),
    )(hpre, degp)


def _tc_elem(accp, hs, dinv, b2d):
    def body(accp_ref, hs_ref, dinv_ref, b_ref, out_ref):
        acc = jnp.concatenate([accp_ref[0], accp_ref[1]], axis=1)
        h = jnp.maximum(dinv_ref[...] * (acc + hs_ref[...]) + b_ref[...], 0.0)
        out_ref[...] = dinv_ref[...] * h

    return pl.pallas_call(
        body,
        out_shape=jax.ShapeDtypeStruct((NPAD, hs.shape[1]), jnp.float32),
    )(accp, hs, dinv, b2d)


def _tc_matmul(accp, p, dinv, b2d, w):
    def body(accp_ref, p_ref, dinv_ref, b_ref, w_ref, out_ref):
        acc = jnp.concatenate([accp_ref[0], accp_ref[1]], axis=1)
        t = dinv_ref[...] * (acc + p_ref[...])
        h = jnp.maximum(jnp.dot(t, w_ref[...],
                                preferred_element_type=jnp.float32) + b_ref[...],
                        0.0)
        out_ref[...] = dinv_ref[...] * h

    return pl.pallas_call(
        body,
        out_shape=jax.ShapeDtypeStruct((NPAD, w.shape[1]), jnp.float32),
    )(accp, p, dinv, b2d, w)


_RB = 1024
_GRID = NPAD // _RB


def _tc_head(accp, p3, dinv, w3, b3_2d, batch2d, wl1, bl1, wl2, bl2, wl3, bl3):
    def body(accp_ref, p_ref, dinv_ref, w3_ref, b3_ref, batch_ref, wl1_ref,
             bl1_ref, wl2_ref, bl2_ref, wl3_ref, bl3_ref, out_ref, sums, cnts):
        i = pl.program_id(0)
        acc = jnp.concatenate([accp_ref[0], accp_ref[1]], axis=1)
        t3 = dinv_ref[...] * (acc + p_ref[...])
        h3 = jnp.dot(t3, w3_ref[...],
                     preferred_element_type=jnp.float32) + b3_ref[...]
        z = jnp.maximum(
            jnp.dot(h3.astype(jnp.bfloat16), wl1_ref[...].astype(jnp.bfloat16),
                    preferred_element_type=jnp.float32) + bl1_ref[...], 0.0)
        z = jnp.maximum(
            jnp.dot(z.astype(jnp.bfloat16), wl2_ref[...].astype(jnp.bfloat16),
                    preferred_element_type=jnp.float32) + bl2_ref[...], 0.0)
        gids = lax.broadcasted_iota(jnp.int32, (NG, _RB), 0)
        onehot = (gids == jnp.broadcast_to(batch_ref[...], (NG, _RB))).astype(jnp.float32)
        psum = jnp.dot(onehot, z, preferred_element_type=jnp.float32)
        pcnt = jnp.sum(onehot, axis=1, keepdims=True)

        @pl.when(i == 0)
        def _():
            sums[...] = jnp.zeros_like(sums)
            cnts[...] = jnp.zeros_like(cnts)

        sums[...] += psum
        cnts[...] += pcnt

        @pl.when(i == _GRID - 1)
        def _():
            mean = sums[...] / jnp.maximum(cnts[...], 1.0)
            out_ref[...] = jnp.dot(mean, wl3_ref[...],
                                   preferred_element_type=jnp.float32) + bl3_ref[...]

    h2 = wl2.shape[0]
    h3w = wl2.shape[1]
    return pl.pallas_call(
        body,
        grid=(_GRID,),
        in_specs=[
            pl.BlockSpec((2, _RB, 32), lambda i: (0, i, 0)),
            pl.BlockSpec((_RB, 64), lambda i: (i, 0)),
            pl.BlockSpec((_RB, 1), lambda i: (i, 0)),
            pl.BlockSpec((64, 128), lambda i: (0, 0)),
            pl.BlockSpec((1, 128), lambda i: (0, 0)),
            pl.BlockSpec((1, _RB), lambda i: (0, i)),
            pl.BlockSpec((128, h2), lambda i: (0, 0)),
            pl.BlockSpec((1, h2), lambda i: (0, 0)),
            pl.BlockSpec((h2, h3w), lambda i: (0, 0)),
            pl.BlockSpec((1, h3w), lambda i: (0, 0)),
            pl.BlockSpec((h3w, 4), lambda i: (0, 0)),
            pl.BlockSpec((1, 4), lambda i: (0, 0)),
        ],
        out_specs=pl.BlockSpec((NG, 4), lambda i: (0, 0)),
        out_shape=jax.ShapeDtypeStruct((NG, 4), jnp.float32),
        scratch_shapes=[
            pltpu.VMEM((NG, h3w), jnp.float32),
            pltpu.VMEM((NG, 1), jnp.float32),
        ],
    )(accp, p3, dinv, w3, b3_2d, batch2d, wl1, bl1, wl2, bl2, wl3, bl3)


def kernel(x, edge_index, batch, W1, b1, W2, b2, W3, b3,
           Wl1, bl1, Wl2, bl2, Wl3, bl3):
    epad = jnp.full((2, _EPAD - E), NPAD - 1, jnp.int32)
    em = jnp.concatenate([edge_index.astype(jnp.int32), epad], axis=1)
    srcm = em[0].reshape(_EROWS, _ECH)
    dstm = em[1].reshape(_EROWS, _ECH)
    xp = jnp.pad(x, ((0, NPAD - N), (0, 0)))
    batch2d = jnp.pad(batch.astype(jnp.int32), (0, NPAD - N),
                      constant_values=NG).reshape(1, NPAD)

    hpre1 = _tc_xw(xp, W1)
    degp = _sc_degree(dstm).reshape(_NC, NPAD, 1)
    dinv, hs1 = _tc_scale(hpre1, degp)
    acc1 = _spmm16(srcm, dstm, hs1[:, :16], hs1[:, 16:])
    p2 = _tc_elem(acc1, hs1, dinv, b1.reshape(1, -1))
    acc2 = _spmm16(srcm, dstm, p2[:, :16], p2[:, 16:])
    p3 = _tc_matmul(acc2, p2, dinv, b2.reshape(1, -1), W2)
    acc3 = _spmm32(srcm, dstm, p3[:, :32], p3[:, 32:])
    return _tc_head(acc3, p3, dinv, W3, b3.reshape(1, -1), batch2d,
                    Wl1, bl1.reshape(1, -1), Wl2, bl2.reshape(1, -1),
                    Wl3, bl3.reshape(1, -1))


# speedup vs baseline: 23.4291x; 1.0477x over previous
import functools

import jax
import jax.numpy as jnp
from jax import lax
from jax.experimental import pallas as pl
from jax.experimental.pallas import tpu as pltpu
from jax.experimental.pallas import tpu_sc as plsc

N = 10000
NPAD = 10240
E = 320000
NG = 64

_NC = 2
_NS = 16
_NW = _NC * _NS
_ECH = 128
_EROWS = 2560
_EPAD = _ECH * _EROWS
_RPW = _EROWS // _NW
_RPS = NPAD // _NS

_mesh = plsc.VectorSubcoreMesh(core_axis_name="c", subcore_axis_name="s")
_sc_params = pltpu.CompilerParams(use_tc_tiling_on_sc=False)

_NB = 8
_EC2 = 2 * _ECH
_NPP = _EROWS // 2 // _NS


def _zero_fill(buf, rows, width):
    zero16 = jnp.zeros((16,), jnp.float32)

    def _z(r, _):
        for j in range(width // 16):
            buf[r, pl.ds(j * 16, 16)] = zero16
        return 0

    lax.fori_loop(0, rows, _z, 0)


def _edge_pipeline(hs_hbm, srcv, dstv, bufs, gsems, ssems, acc_sh, nrows):
    for j in range(_NB):
        pltpu.async_copy(hs_hbm.at[srcv.at[j]], bufs[j], gsems[j])

    def _grp(i, _):
        k0 = _NB * i
        for j in range(_NB):
            pltpu.make_async_copy(hs_hbm.at[srcv.at[k0 + j]], bufs[j],
                                  gsems[j]).wait()
            pltpu.async_copy(bufs[j], acc_sh.at[dstv.at[k0 + j]], ssems[j],
                             add=True)
        for j in range(_NB):
            nxt = k0 + _NB + j
            pltpu.make_async_copy(bufs[j], acc_sh.at[dstv.at[k0 + j]],
                                  ssems[j]).wait()

            @pl.when(nxt < nrows)
            def _():
                pltpu.async_copy(hs_hbm.at[srcv.at[nxt]], bufs[j], gsems[j])

        return 0

    lax.fori_loop(0, nrows // _NB, _grp, 0)


@functools.partial(
    pl.kernel,
    out_type=jax.ShapeDtypeStruct((_NC, NPAD), jnp.float32),
    mesh=_mesh,
    scratch_types=[
        pltpu.VMEM((_RPW, _ECH), jnp.int32),
        pltpu.VMEM((_ECH,), jnp.float32),
        pltpu.VMEM((_RPS,), jnp.float32),
        pltpu.VMEM_SHARED((NPAD,), jnp.float32),
        pltpu.SemaphoreType.DMA,
    ],
    compiler_params=_sc_params,
)
def _sc_degree(dstm_hbm, out_hbm, dstv, ones_v, zrow_v, acc_sh, sem):
    c = lax.axis_index("c")
    s = lax.axis_index("s")
    w = s * _NC + c
    one16 = jnp.ones((16,), jnp.float32)
    zero16 = jnp.zeros((16,), jnp.float32)
    for i in range(_ECH // 16):
        ones_v[pl.ds(i * 16, 16)] = one16

    def _z(i, _):
        zrow_v[pl.ds(i * 16, 16)] = zero16
        return 0

    lax.fori_loop(0, _RPS // 16, _z, 0)
    pltpu.sync_copy(zrow_v, acc_sh.at[pl.ds(s * _RPS, _RPS)])
    pltpu.sync_copy(dstm_hbm.at[pl.ds(w * _RPW, _RPW)], dstv)
    plsc.subcore_barrier()

    def _body(i, _):
        for j in range(8):
            pltpu.async_copy(ones_v, acc_sh.at[dstv.at[8 * i + j]],
                             sem, add=True)
        for j in range(8):
            pltpu.make_async_copy(ones_v, acc_sh.at[dstv.at[8 * i + j]],
                                  sem).wait()
        return 0

    lax.fori_loop(0, _RPW // 8, _body, 0)
    plsc.subcore_barrier()
    pltpu.sync_copy(acc_sh.at[pl.ds(s * _RPS, _RPS)],
                    out_hbm.at[c, pl.ds(s * _RPS, _RPS)])


_RPC = _EROWS // _NS


def _make_spmm(FH):
    @functools.partial(
        pl.kernel,
        out_type=jax.ShapeDtypeStruct((_NC, NPAD, FH), jnp.float32),
        mesh=_mesh,
        scratch_types=[
            pltpu.VMEM((_RPC, _ECH), jnp.int32),
            pltpu.VMEM((_RPC, _ECH), jnp.int32),
            [pltpu.VMEM((_ECH, FH), jnp.float32)] * _NB,
            pltpu.VMEM_SHARED((NPAD, FH), jnp.float32),
            [pltpu.SemaphoreType.DMA] * _NB,
            [pltpu.SemaphoreType.DMA] * _NB,
        ],
        compiler_params=_sc_params,
    )
    def _spmm(srcm_hbm, dstm_hbm, hsa_hbm, hsb_hbm, out_hbm, srcv, dstv,
              bufs, acc_sh, gsems, ssems):
        c = lax.axis_index("c")
        s = lax.axis_index("s")
        _zero_fill(bufs[0], _ECH, FH)
        for t in range(_RPS // _ECH):
            pltpu.sync_copy(bufs[0], acc_sh.at[pl.ds(s * _RPS + t * _ECH, _ECH)])
        pltpu.sync_copy(srcm_hbm.at[pl.ds(s * _RPC, _RPC)], srcv)
        pltpu.sync_copy(dstm_hbm.at[pl.ds(s * _RPC, _RPC)], dstv)
        plsc.subcore_barrier()

        @pl.when(c == 0)
        def _():
            _edge_pipeline(hsa_hbm, srcv, dstv, bufs, gsems, ssems, acc_sh, _RPC)

        @pl.when(c == 1)
        def _():
            _edge_pipeline(hsb_hbm, srcv, dstv, bufs, gsems, ssems, acc_sh, _RPC)

        plsc.subcore_barrier()
        pltpu.sync_copy(acc_sh.at[pl.ds(s * _RPS, _RPS)],
                        out_hbm.at[c, pl.ds(s * _RPS, _RPS)])

    return _spmm


_spmm16 = _make_spmm(16)
_spmm32 = _make_spmm(32)


def _tc_xw(xp, w1):
    def body(x_ref, w1_ref, out_ref):
        out_ref[...] = jnp.dot(x_ref[...], w1_ref[...],
                               preferred_element_type=jnp.float32)

    return pl.pallas_call(
        body,
        out_shape=jax.ShapeDtypeStruct((NPAD, w1.shape[1]), jnp.float32),
    )(xp, w1)


def _tc_scale(hpre, degp):
    def body(hpre_ref, degp_ref, dinv_ref, hs_ref):
        deg = degp_ref[0] + degp_ref[1] + 1.0
        dinv = lax.rsqrt(deg)
        dinv_ref[...] = dinv
        hs_ref[...] = dinv * hpre_ref[...]

    return pl.pallas_call(
        body,
        out_shape=(
            jax.ShapeDtypeStruct((NPAD, 1), jnp.float32),
            jax.ShapeDtypeStruct((NPAD, hpre.shape[1]), jnp.float32),
        ),
    )(hpre, degp)


def _tc_elem(accp, hs, dinv, b2d):
    def body(accp_ref, hs_ref, dinv_ref, b_ref, out_ref):
        acc = jnp.concatenate([accp_ref[0], accp_ref[1]], axis=1)
        h = jnp.maximum(dinv_ref[...] * (acc + hs_ref[...]) + b_ref[...], 0.0)
        out_ref[...] = dinv_ref[...] * h

    return pl.pallas_call(
        body,
        out_shape=jax.ShapeDtypeStruct((NPAD, hs.shape[1]), jnp.float32),
    )(accp, hs, dinv, b2d)


def _tc_matmul(accp, p, dinv, b2d, w):
    def body(accp_ref, p_ref, dinv_ref, b_ref, w_ref, out_ref):
        acc = jnp.concatenate([accp_ref[0], accp_ref[1]], axis=1)
        t = dinv_ref[...] * (acc + p_ref[...])
        h = jnp.maximum(jnp.dot(t, w_ref[...],
                                preferred_element_type=jnp.float32) + b_ref[...],
                        0.0)
        out_ref[...] = dinv_ref[...] * h

    return pl.pallas_call(
        body,
        out_shape=jax.ShapeDtypeStruct((NPAD, w.shape[1]), jnp.float32),
    )(accp, p, dinv, b2d, w)


_RB = 1024
_GRID = NPAD // _RB


def _tc_head(accp, p3, dinv, w3, b3_2d, batch2d, wl1, bl1, wl2, bl2, wl3, bl3):
    def body(accp_ref, p_ref, dinv_ref, w3_ref, b3_ref, batch_ref, wl1_ref,
             bl1_ref, wl2_ref, bl2_ref, wl3_ref, bl3_ref, out_ref, sums, cnts):
        i = pl.program_id(0)
        acc = jnp.concatenate([accp_ref[0], accp_ref[1]], axis=1)
        t3 = dinv_ref[...] * (acc + p_ref[...])
        h3 = jnp.dot(t3, w3_ref[...],
                     preferred_element_type=jnp.float32) + b3_ref[...]
        z = jnp.maximum(
            jnp.dot(h3.astype(jnp.bfloat16), wl1_ref[...].astype(jnp.bfloat16),
                    preferred_element_type=jnp.float32) + bl1_ref[...], 0.0)
        z = jnp.maximum(
            jnp.dot(z.astype(jnp.bfloat16), wl2_ref[...].astype(jnp.bfloat16),
                    preferred_element_type=jnp.float32) + bl2_ref[...], 0.0)
        gids = lax.broadcasted_iota(jnp.int32, (NG, _RB), 0)
        onehot = (gids == jnp.broadcast_to(batch_ref[...], (NG, _RB))).astype(jnp.float32)
        psum = jnp.dot(onehot, z, preferred_element_type=jnp.float32)
        pcnt = jnp.sum(onehot, axis=1, keepdims=True)

        @pl.when(i == 0)
        def _():
            sums[...] = jnp.zeros_like(sums)
            cnts[...] = jnp.zeros_like(cnts)

        sums[...] += psum
        cnts[...] += pcnt

        @pl.when(i == _GRID - 1)
        def _():
            mean = sums[...] / jnp.maximum(cnts[...], 1.0)
            out_ref[...] = jnp.dot(mean, wl3_ref[...],
                                   preferred_element_type=jnp.float32) + bl3_ref[...]

    h2 = wl2.shape[0]
    h3w = wl2.shape[1]
    return pl.pallas_call(
        body,
        grid=(_GRID,),
        in_specs=[
            pl.BlockSpec((2, _RB, 32), lambda i: (0, i, 0)),
            pl.BlockSpec((_RB, 64), lambda i: (i, 0)),
            pl.BlockSpec((_RB, 1), lambda i: (i, 0)),
            pl.BlockSpec((64, 128), lambda i: (0, 0)),
            pl.BlockSpec((1, 128), lambda i: (0, 0)),
            pl.BlockSpec((1, _RB), lambda i: (0, i)),
            pl.BlockSpec((128, h2), lambda i: (0, 0)),
            pl.BlockSpec((1, h2), lambda i: (0, 0)),
            pl.BlockSpec((h2, h3w), lambda i: (0, 0)),
            pl.BlockSpec((1, h3w), lambda i: (0, 0)),
            pl.BlockSpec((h3w, 4), lambda i: (0, 0)),
            pl.BlockSpec((1, 4), lambda i: (0, 0)),
        ],
        out_specs=pl.BlockSpec((NG, 4), lambda i: (0, 0)),
        out_shape=jax.ShapeDtypeStruct((NG, 4), jnp.float32),
        scratch_shapes=[
            pltpu.VMEM((NG, h3w), jnp.float32),
            pltpu.VMEM((NG, 1), jnp.float32),
        ],
    )(accp, p3, dinv, w3, b3_2d, batch2d, wl1, bl1, wl2, bl2, wl3, bl3)


def kernel(x, edge_index, batch, W1, b1, W2, b2, W3, b3,
           Wl1, bl1, Wl2, bl2, Wl3, bl3):
    epad = jnp.full((2, _EPAD - E), NPAD - 1, jnp.int32)
    em = jnp.concatenate([edge_index.astype(jnp.int32), epad], axis=1)
    srcm = em[0].reshape(_EROWS, _ECH)
    dstm = em[1].reshape(_EROWS, _ECH)
    xp = jnp.pad(x, ((0, NPAD - N), (0, 0)))
    batch2d = jnp.pad(batch.astype(jnp.int32), (0, NPAD - N),
                      constant_values=NG).reshape(1, NPAD)

    hpre1 = _tc_xw(xp, W1)
    degp = _sc_degree(dstm).reshape(_NC, NPAD, 1)
    dinv, hs1 = _tc_scale(hpre1, degp)
    acc1 = _spmm16(srcm, dstm, hs1[:, :16], hs1[:, 16:])
    p2 = _tc_elem(acc1, hs1, dinv, b1.reshape(1, -1))
    acc2 = _spmm16(srcm, dstm, p2[:, :16], p2[:, 16:])
    p3 = _tc_matmul(acc2, p2, dinv, b2.reshape(1, -1), W2)
    acc3 = _spmm32(srcm, dstm, p3[:, :32], p3[:, 32:])
    return _tc_head(acc3, p3, dinv, W3, b3.reshape(1, -1), batch2d,
                    Wl1, bl1.reshape(1, -1), Wl2, bl2.reshape(1, -1),
                    Wl3, bl3.reshape(1, -1))
